# baseline (speedup 1.0000x reference)
import numpy as np
import jax
import jax.numpy as jnp
from jax.experimental import pallas as pl

_LEVEL_SIZES = [64, 32, 16, 8, 4]
_STRIDES = [8, 16, 32, 64, 128]
_C = 80
_B = 8
_N = sum(hw * hw for hw in _LEVEL_SIZES)


def _coords_np():
    xs, ys = [], []
    for hw, s in zip(_LEVEL_SIZES, _STRIDES):
        ii, jj = np.meshgrid(np.arange(hw), np.arange(hw), indexing="ij")
        xs.append((jj.reshape(-1) + 0.5) * s)
        ys.append((ii.reshape(-1) + 0.5) * s)
    return (np.concatenate(xs).astype(np.float32),
            np.concatenate(ys).astype(np.float32))


_XC_NP, _YC_NP = _coords_np()
_NR, _NL = 8, _N // 8


def _fcos_body(conf_ref, cls_ref, loc_ref, bd_ref, cen_ref, ct_ref,
               posf_ref, xy_ref, out_ref):
    b = pl.program_id(0)

    conf = jnp.clip(conf_ref[0], 1e-08, 0.99999999)
    one_m = 1.0 - conf
    neg_term = -0.75 * conf * conf * jnp.log(one_m)
    pos_term = -0.25 * one_m * one_m * jnp.log(conf)
    iota = jax.lax.broadcasted_iota(jnp.int32, (_N, _C), 1)
    onehot = cls_ref[0] == iota
    loss_conf = jnp.sum(jnp.where(onehot, pos_term, neg_term))

    x = xy_ref[0]
    y = xy_ref[1]
    loc = loc_ref[0] * 64.0
    bd = bd_ref[0] * 64.0
    l1 = x - loc[0]
    t1 = y - loc[1]
    r1 = x + loc[2]
    b1 = y + loc[3]
    l2 = x - bd[0]
    t2 = y - bd[1]
    r2 = x + bd[2]
    b2 = y + bd[3]
    s1 = (b1 - t1 + 1.0) * (r1 - l1 + 1.0)
    s2 = (b2 - t2 + 1.0) * (r2 - l2 + 1.0)
    cl = jnp.maximum(l1, l2)
    cr = jnp.minimum(r1, r2)
    ct = jnp.maximum(t1, t2)
    cb = jnp.minimum(b1, b2)
    s_cross = (cr - cl + 1.0) * (cb - ct + 1.0)
    union = s1 + s2 - s_cross
    valid = (cl < cr) & (ct < cb) & (union > 0) & (s_cross > 0)
    ratio = jnp.where(valid, s_cross / jnp.where(valid, union, 1.0), 1.0)
    iou_l = jnp.where(valid, -jnp.log(ratio), 0.0)

    posf = posf_ref[0]
    loss_l = jnp.sum(iou_l * posf)

    p = jnp.clip(cen_ref[0], 1e-07, 1.0 - 1e-07)
    tgt = ct_ref[0]
    bce = -(tgt * jnp.log(p) + (1.0 - tgt) * jnp.log(1.0 - p))
    loss_center = jnp.sum(bce * posf)

    poses = jnp.sum(posf)
    denom = jnp.where(poses > 0, poses, 1.0)
    per_img = jnp.where(poses > 0,
                        loss_center + (loss_conf + loss_l) / denom,
                        loss_center + loss_conf + loss_l)

    contrib = jnp.full((1, 1), per_img / _B, dtype=jnp.float32)

    @pl.when(b == 0)
    def _init():
        out_ref[...] = contrib

    @pl.when(b != 0)
    def _acc():
        out_ref[...] += contrib


def kernel(confs, locs, centers, box_dists, center_targets, cls_targets,
           pos_mask):
    cls3 = cls_targets.astype(jnp.int32)[..., None]
    loc4 = jnp.moveaxis(locs, -1, 1).reshape(_B, 4, _NR, _NL)
    bd4 = jnp.moveaxis(box_dists, -1, 1).reshape(_B, 4, _NR, _NL)
    cen = centers.reshape(_B, _NR, _NL)
    ctg = center_targets.reshape(_B, _NR, _NL)
    posf = (pos_mask == 1).astype(jnp.float32).reshape(_B, _NR, _NL)
    xy = jnp.asarray(np.stack([_XC_NP.reshape(_NR, _NL),
                               _YC_NP.reshape(_NR, _NL)]))

    out = pl.pallas_call(
        _fcos_body,
        grid=(_B,),
        in_specs=[
            pl.BlockSpec((1, _N, _C), lambda b: (b, 0, 0)),
            pl.BlockSpec((1, _N, 1), lambda b: (b, 0, 0)),
            pl.BlockSpec((1, 4, _NR, _NL), lambda b: (b, 0, 0, 0)),
            pl.BlockSpec((1, 4, _NR, _NL), lambda b: (b, 0, 0, 0)),
            pl.BlockSpec((1, _NR, _NL), lambda b: (b, 0, 0)),
            pl.BlockSpec((1, _NR, _NL), lambda b: (b, 0, 0)),
            pl.BlockSpec((1, _NR, _NL), lambda b: (b, 0, 0)),
            pl.BlockSpec((2, _NR, _NL), lambda b: (0, 0, 0)),
        ],
        out_specs=pl.BlockSpec((1, 1), lambda b: (0, 0)),
        out_shape=jax.ShapeDtypeStruct((1, 1), jnp.float32),
    )(confs, cls3, loc4, bd4, cen, ctg, posf, xy)
    return out[0, 0]
